# baseline (speedup 1.0000x reference)
import functools

import jax
import jax.numpy as jnp
from jax import lax
from jax.experimental import pallas as pl
from jax.experimental.pallas import tpu as pltpu
from jax.experimental.pallas import tpu_sc as plsc

_B = 32
_N = 17
_HW = 16384
_L = 16
_NCHUNK = _HW // _L
_GTP = 48


def _make_sc_call():
    mesh = plsc.VectorSubcoreMesh(core_axis_name="c", subcore_axis_name="s")

    @functools.partial(
        pl.kernel,
        mesh=mesh,
        out_type=jax.ShapeDtypeStruct((_B * _L,), jnp.float32),
        scratch_types=[
            pltpu.VMEM((_HW,), jnp.float32),
            pltpu.VMEM((_HW,), jnp.float32),
            pltpu.VMEM((2 * _HW + _L,), jnp.float32),
            pltpu.VMEM((_GTP,), jnp.float32),
            pltpu.VMEM((_L,), jnp.float32),
            pltpu.SemaphoreType.DMA,
            pltpu.SemaphoreType.DMA,
            pltpu.SemaphoreType.DMA,
        ],
    )
    def sc_loss(hm_hbm, off_hbm, gt_hbm, out_hbm,
                row_a, row_b, off_v, gt_v, part_v,
                sem_a, sem_b, sem_c):
        w = lax.axis_index("s") * 2 + lax.axis_index("c")

        off_cp = pltpu.async_copy(
            off_hbm.at[pl.ds(w * (2 * _HW), 2 * _HW)],
            off_v.at[pl.ds(0, 2 * _HW)],
            sem_c,
        )
        pltpu.sync_copy(gt_hbm.at[pl.ds(w * _GTP, _GTP)], gt_v)

        hm_base = w * (_N * _HW)
        bufs = (row_a, row_b)
        sems = (sem_a, sem_b)
        copies = [None, None]
        copies[0] = pltpu.async_copy(
            hm_hbm.at[pl.ds(hm_base, _HW)], row_a, sems[0]
        )

        lane = lax.broadcasted_iota(jnp.int32, (_L,), 0)
        ox = jnp.zeros((_L,), jnp.float32)
        oy = jnp.zeros((_L,), jnp.float32)
        gx = jnp.zeros((_L,), jnp.float32)
        gy = jnp.zeros((_L,), jnp.float32)
        off_waited = False

        for k in range(_N):
            buf = bufs[k % 2]
            copies[k % 2].wait()
            if k + 1 < _N:
                copies[(k + 1) % 2] = pltpu.async_copy(
                    hm_hbm.at[pl.ds(hm_base + (k + 1) * _HW, _HW)],
                    bufs[(k + 1) % 2],
                    sems[(k + 1) % 2],
                )

            _Q = 4
            _QLEN = _NCHUNK // _Q

            def chunk_body(jj, carry, buf=buf):
                new = []
                for q in range(_Q):
                    run_max, run_j = carry[2 * q], carry[2 * q + 1]
                    v = buf[pl.ds((q * _QLEN + jj) * _L, _L)]
                    upd = v > run_max
                    new.append(jnp.where(upd, v, run_max))
                    new.append(jnp.where(upd, jj, run_j))
                return tuple(new)

            init_q = []
            for _ in range(_Q):
                init_q.append(jnp.full((_L,), -jnp.inf, jnp.float32))
                init_q.append(jnp.zeros((_L,), jnp.int32))
            acc = lax.fori_loop(0, _QLEN, chunk_body, tuple(init_q), unroll=4)

            best_v = acc[0]
            best_f = (acc[1] * _L) + lane
            for q in range(1, _Q):
                o_v = acc[2 * q]
                o_f = (q * _QLEN + acc[2 * q + 1]) * _L + lane
                upd = (o_v > best_v) | ((o_v == best_v) & (o_f < best_f))
                best_v = jnp.where(upd, o_v, best_v)
                best_f = jnp.where(upd, o_f, best_f)

            for s in (8, 4, 2, 1):
                perm = lane ^ s
                o_v = best_v.at[perm].get(mode="promise_in_bounds")
                o_f = best_f.at[perm].get(mode="promise_in_bounds")
                upd = (o_v > best_v) | ((o_v == best_v) & (o_f < best_f))
                best_v = jnp.where(upd, o_v, best_v)
                best_f = jnp.where(upd, o_f, best_f)

            idx_k = best_f[0]

            if not off_waited:
                off_cp.wait()
                off_waited = True
            ox_k = off_v[pl.ds(idx_k, _L)][0]
            oy_k = off_v[pl.ds(idx_k + _HW, _L)][0]
            gvec = gt_v[pl.ds(2 * k, _L)]
            gx_k = gvec[0]
            gy_k = gvec[1]
            tgt = k % _L
            ox = jnp.where(lane == tgt, ox_k, ox) if k < _L else ox
            oy = jnp.where(lane == tgt, oy_k, oy) if k < _L else oy
            gx = jnp.where(lane == tgt, gx_k, gx) if k < _L else gx
            gy = jnp.where(lane == tgt, gy_k, gy) if k < _L else gy
            if k >= _L:
                extra = (k, ox_k, oy_k, gx_k, gy_k)

        err = jnp.abs(ox - gx) + jnp.abs(oy - gy)
        _, eox, eoy, egx, egy = extra
        err_extra = jnp.abs(eox - egx) + jnp.abs(eoy - egy)
        err = err + jnp.where(lane == 0, err_extra, 0.0)
        part_v[...] = err
        pltpu.sync_copy(part_v, out_hbm.at[pl.ds(w * _L, _L)])

    return sc_loss


_sc_call = _make_sc_call()


@jax.jit
def _run(hm_flat, off_flat, gt_pad):
    parts = _sc_call(hm_flat, off_flat, gt_pad)
    return jnp.sum(parts) * (1.0 / (_B * _N * 2 * _N))


def kernel(offset_map_pred, hm_gt, offset_gt):
    b, n = hm_gt.shape[0], hm_gt.shape[1]
    hm_flat = hm_gt.reshape(-1)
    off_flat = offset_map_pred.reshape(-1)
    gt_pad = jnp.zeros((b, _GTP), jnp.float32)
    gt_pad = gt_pad.at[:, : 2 * n].set(offset_gt.reshape(b, 2 * n))
    return _run(hm_flat, off_flat, gt_pad.reshape(-1))
